# baseline (speedup 1.0000x reference)
import functools

import jax
import jax.numpy as jnp
from jax import lax
from jax.experimental import pallas as pl
from jax.experimental.pallas import tpu as pltpu
from jax.experimental.pallas import tpu_sc as plsc

N_ITEM = 40000
N_USER = 10000
N = N_ITEM + N_USER
D = 128
NNZ = 200000

NC = 2
NS = 16
L = 16

E_TOT = 3 * NNZ
E_PAD = 614400
CHUNK = E_PAD // NS
EB = 128
NBLK = CHUNK // EB
R = 12800
N_PASS = 2
ACC_ROWS = 13312
MSG_PAD = N_PASS * NC * R


def _mm3_body(h_ref, w_ref, o_ref):
    o_ref[...] = lax.dot_general(
        h_ref[...], w_ref[0],
        (((1,), (1,)), ((), ())),
        preferred_element_type=jnp.float32)


def _transform_all(h_all, w_stack):
    blk = 1000
    nb = N // blk
    return pl.pallas_call(
        _mm3_body,
        grid=(3, nb),
        in_specs=[
            pl.BlockSpec((blk, D), lambda r, i: (i, 0)),
            pl.BlockSpec((1, D, D), lambda r, i: (r, 0, 0)),
        ],
        out_specs=pl.BlockSpec((blk, D), lambda r, i: (r * nb + i, 0)),
        out_shape=jax.ShapeDtypeStruct((3 * N, D), jnp.float32),
    )(h_all, w_stack)


def _sc_body(x_hbm, src_hbm, dst_hbm, val_hbm, msg_hbm,
             acc, rows, srcb, dstb, valb, dloc, sem):
    c = lax.axis_index("c")
    s = lax.axis_index("s")
    iota = lax.iota(jnp.int32, L)
    lane_ids = [jnp.full((L, 1), l, dtype=jnp.int32) for l in range(L)]
    gdn = lax.GatherDimensionNumbers(
        offset_dims=(), collapsed_slice_dims=(0,), start_index_map=(0,))

    def _lane_bcast(v, l):
        return lax.gather(v, lane_ids[l], gdn, slice_sizes=(1,),
                          mode=lax.GatherScatterMode.PROMISE_IN_BOUNDS)

    for p in range(N_PASS):
        lo = (N_PASS * c + p) * R

        def _zero_rows(i, carry):
            for q in range(8):
                rows[i, pl.ds(q * L, L)] = jnp.zeros((L,), jnp.float32)
            return carry
        lax.fori_loop(0, EB, _zero_rows, 0)
        tile_rows = ACC_ROWS // NS
        for z in range(tile_rows // EB):
            pltpu.sync_copy(rows, acc.at[pl.ds(s * tile_rows + z * EB, EB)])
        rem = tile_rows % EB
        if rem:
            pltpu.sync_copy(rows.at[pl.ds(0, rem)],
                            acc.at[pl.ds(s * tile_rows + tile_rows - rem, rem)])
        plsc.subcore_barrier()

        def _block(b, carry):
            e0 = s * CHUNK + b * EB
            pltpu.sync_copy(src_hbm.at[pl.ds(e0, EB)], srcb)
            pltpu.sync_copy(dst_hbm.at[pl.ds(e0, EB)], dstb)
            pltpu.sync_copy(val_hbm.at[pl.ds(e0, EB)], valb)
            pltpu.async_copy(x_hbm.at[srcb], rows, sem).wait()

            def _group(i, carry2):
                v_dst = dstb[pl.ds(i * L, L)]
                v_val = valb[pl.ds(i * L, L)]
                m = (v_dst >= lo) & (v_dst < lo + R)
                v_val = jnp.where(m, v_val, 0.0)
                dloc[0, pl.ds(i * L, L)] = jnp.where(m, v_dst - lo, R + iota)
                for l in range(L):
                    bc = _lane_bcast(v_val, l)
                    r = i * L + l
                    for q in range(8):
                        rows[r, pl.ds(q * L, L)] = rows[r, pl.ds(q * L, L)] * bc
                return carry2
            lax.fori_loop(0, EB // L, _group, 0)
            pltpu.sync_copy(rows, acc.at[dloc.at[0]], add=True)
            return carry
        lax.fori_loop(0, NBLK, _block, 0)
        plsc.subcore_barrier()

        rows_per_tile = R // NS
        pltpu.sync_copy(acc.at[pl.ds(s * rows_per_tile, rows_per_tile)],
                        msg_hbm.at[pl.ds(lo + s * rows_per_tile, rows_per_tile)])
        plsc.subcore_barrier()


def _message_pass(x, src, dst, val):
    mesh = plsc.VectorSubcoreMesh(core_axis_name="c", subcore_axis_name="s",
                                  num_cores=NC, num_subcores=NS)
    f = pl.kernel(
        _sc_body,
        out_type=jax.ShapeDtypeStruct((MSG_PAD, D), jnp.float32),
        mesh=mesh,
        scratch_types=[
            pltpu.VMEM_SHARED((ACC_ROWS, D), jnp.float32),
            pltpu.VMEM((EB, D), jnp.float32),
            pltpu.VMEM((EB,), jnp.int32),
            pltpu.VMEM((EB,), jnp.int32),
            pltpu.VMEM((EB,), jnp.float32),
            pltpu.VMEM((1, EB), jnp.int32),
            pltpu.SemaphoreType.DMA,
        ],
    )
    return f(x, src, dst, val)


def _out_body(m_ref, h_ref, wi_ref, wu_ref, bi_ref, bu_ref, o_ref):
    i = pl.program_id(0)
    z = m_ref[...] + h_ref[...]

    def _apply(w, b):
        o_ref[...] = jnp.maximum(
            lax.dot_general(z, w, (((1,), (1,)), ((), ())),
                            preferred_element_type=jnp.float32) + b, 0.0)

    pl.when(i < N_ITEM // 400)(lambda: _apply(wi_ref[...], bi_ref[...]))
    pl.when(i >= N_ITEM // 400)(lambda: _apply(wu_ref[...], bu_ref[...]))


def _output_layer(msg, h_all, w_item, b_item, w_user, b_user):
    blk = 400
    return pl.pallas_call(
        _out_body,
        grid=(N // blk,),
        in_specs=[
            pl.BlockSpec((blk, D), lambda i: (i, 0)),
            pl.BlockSpec((blk, D), lambda i: (i, 0)),
            pl.BlockSpec((D, D), lambda i: (0, 0)),
            pl.BlockSpec((D, D), lambda i: (0, 0)),
            pl.BlockSpec((1, D), lambda i: (0, 0)),
            pl.BlockSpec((1, D), lambda i: (0, 0)),
        ],
        out_specs=pl.BlockSpec((blk, D), lambda i: (i, 0)),
        out_shape=jax.ShapeDtypeStruct((N, D), jnp.float32),
    )(msg, h_all, w_item, w_user, b_item.reshape(1, D), b_user.reshape(1, D))


def kernel(h_item, h_user, A0_values, A1_values, A2_values,
           W_r0, W_r1, W_r2, W_item, b_item, W_user, b_user,
           A0_indices, A1_indices, A2_indices):
    h_all = jnp.concatenate([h_item, h_user], axis=0)
    w_stack = jnp.stack([W_r0, W_r1, W_r2], axis=0)

    x = _transform_all(h_all, w_stack)

    src = jnp.concatenate([
        A0_indices[1].astype(jnp.int32),
        A1_indices[1].astype(jnp.int32) + N,
        A2_indices[1].astype(jnp.int32) + 2 * N,
    ])
    dst = jnp.concatenate([
        A0_indices[0].astype(jnp.int32),
        A1_indices[0].astype(jnp.int32),
        A2_indices[0].astype(jnp.int32),
    ])
    val = jnp.concatenate([A0_values, A1_values, A2_values])
    pad = E_PAD - E_TOT
    src = jnp.concatenate([src, jnp.zeros((pad,), jnp.int32)])
    dst = jnp.concatenate([dst, jnp.zeros((pad,), jnp.int32)])
    val = jnp.concatenate([val, jnp.zeros((pad,), jnp.float32)])

    msg = _message_pass(x, src, dst, val)[:N]

    out = _output_layer(msg, h_all, W_item, b_item, W_user, b_user)
    return (out[:N_ITEM], out[N_ITEM:])
